# baseline (speedup 1.0000x reference)
import jax
import jax.numpy as jnp
from jax.experimental import pallas as pl

N = 10000


def _ln(x, g, b, eps):
    m = x.mean(axis=-1, keepdims=True)
    v = ((x - m) ** 2).mean(axis=-1, keepdims=True)
    return g * (x - m) / jnp.sqrt(v + eps) + b


def _sage(x, ei, Wl, bl, Wr):
    src, dst = ei[0], ei[1]
    agg = jax.ops.segment_sum(x[src], dst, num_segments=N)
    cnt = jax.ops.segment_sum(jnp.ones((ei.shape[1],), jnp.float32), dst, num_segments=N)
    agg = agg / jnp.clip(cnt, 1.0)[:, None]
    return agg @ Wl.T + bl + x @ Wr.T


def _gnorm(x, w, b, a, eps=1e-5):
    mean = x.mean(axis=0, keepdims=True)
    out = x - a * mean
    var = (out * out).mean(axis=0, keepdims=True)
    return w * out / jnp.sqrt(var + eps) + b


def _identity_pallas(x):
    def k(x_ref, o_ref):
        o_ref[...] = x_ref[...]
    return pl.pallas_call(k, out_shape=jax.ShapeDtypeStruct(x.shape, x.dtype))(x)


def kernel(x_res, positions, AA, edge_index_bb, edge_index_knn, params):
    p = params
    ei_a, ei_b = edge_index_bb, edge_index_knn
    g = lambda t: jax.nn.gelu(t, approximate=False)
    x0 = _ln(x_res, p['ln_g'], p['ln_b'], 1e-6)
    h = jax.nn.relu(positions @ p['pW1'].T + p['pb1'])
    h = jax.nn.relu(h @ p['pW2'].T + p['pb2'])
    h = jax.nn.relu(h @ p['pW3'].T + p['pb3'])
    pos_enc = h @ p['pW4'].T + p['pb4']
    xin = jnp.concatenate([x0, pos_enc], axis=1)
    x = _ln(xin, p['ln2_g'], p['ln2_b'], 1e-6)
    x = g(x @ p['iW1'].T + p['ib1'])
    x = g(x @ p['iW2'].T + p['ib2'])
    xa = _sage(x, ei_a, p['Wl_a1'], p['bl_a1'], p['Wr_a1'])
    xb = _sage(x, ei_b, p['Wl_b1'], p['bl_b1'], p['Wr_b1'])
    x = g(0.5 * (xa + xb))
    x = _gnorm(x, p['g1_w'], p['g1_b'], p['g1_a'])
    x1 = x
    xa = _sage(x, ei_a, p['Wl_a2'], p['bl_a2'], p['Wr_a2'])
    xb = _sage(x, ei_b, p['Wl_b2'], p['bl_b2'], p['Wr_b2'])
    x = g(0.5 * (xa + xb))
    x = _gnorm(x, p['g2_w'], p['g2_b'], p['g2_a'])
    x2 = x
    x = jnp.concatenate([x1, x2], axis=1)
    x = g(x @ p['hW1'].T + p['hb1'])
    x = g(x @ p['hW2'].T + p['hb2'])
    x = jnp.concatenate([x, AA], axis=1)
    x = g(x @ p['oW1'].T + p['ob1'])
    x = g(x @ p['oW2'].T + p['ob2'])
    x = jnp.tanh(x @ p['oW3'].T + p['ob3'])
    cb = p['codebook']
    d = jnp.sum(x * x, axis=1, keepdims=True) + jnp.sum(cb * cb, axis=1)[None, :] - 2.0 * (x @ cb.T)
    idx = jnp.argmin(d, axis=1)
    q = jnp.take(cb, idx, axis=0)
    e_loss = jnp.mean((jax.lax.stop_gradient(q) - x) ** 2)
    q_loss = jnp.mean((q - jax.lax.stop_gradient(x)) ** 2)
    vq_loss = q_loss + 0.25 * e_loss
    z_q = x + jax.lax.stop_gradient(q - x)
    z_q = _identity_pallas(z_q)
    return z_q, vq_loss


# speedup vs baseline: 3.9241x; 3.9241x over previous
import functools

import jax
import jax.numpy as jnp
from jax import lax
from jax.experimental import pallas as pl
from jax.experimental.pallas import tpu as pltpu
from jax.experimental.pallas import tpu_sc as plsc

N = 10000
E = 160000
C = 128
NCHUNK = E // C
NSUB = 16
ROWS_PER_TILE = N // NSUB


def _make_seg_sum(width):
    mesh = plsc.VectorSubcoreMesh(core_axis_name="c", subcore_axis_name="s")

    @functools.partial(
        pl.kernel,
        out_type=(jax.ShapeDtypeStruct((N, width), jnp.float32),
                  jax.ShapeDtypeStruct((N, width), jnp.float32)),
        mesh=mesh,
        scratch_types=[
            pltpu.VMEM((C,), jnp.int32),
            pltpu.VMEM((C,), jnp.int32),
            pltpu.VMEM((C, width), jnp.float32),
            pltpu.VMEM_SHARED((N, width), jnp.float32),
            pltpu.SemaphoreType.DMA,
        ],
        compiler_params=pltpu.CompilerParams(use_tc_tiling_on_sc=False),
    )
    def seg(y_a, ei_a, y_b, ei_b, zeros, out_a, out_b, src_v, dst_v, rows_v, acc, sem):
        cid = lax.axis_index("c")
        sid = lax.axis_index("s")
        r0 = sid * ROWS_PER_TILE
        pltpu.sync_copy(zeros.at[pl.ds(r0, ROWS_PER_TILE)], acc.at[pl.ds(r0, ROWS_PER_TILE)])
        plsc.subcore_barrier()

        def edge_loop(y, ei):
            n_i = (NCHUNK - 1 - sid) // NSUB + 1

            def body(i, carry):
                base = (sid + i * NSUB) * C
                pltpu.sync_copy(ei.at[0, pl.ds(base, C)], src_v)
                pltpu.sync_copy(ei.at[1, pl.ds(base, C)], dst_v)
                pltpu.async_copy(y.at[src_v], rows_v, sem).wait()
                pltpu.sync_copy(rows_v, acc.at[dst_v], add=True)
                return carry

            lax.fori_loop(0, n_i, body, 0)

        @pl.when(cid == 0)
        def _():
            edge_loop(y_a, ei_a)

        @pl.when(cid == 1)
        def _():
            edge_loop(y_b, ei_b)

        plsc.subcore_barrier()

        @pl.when(cid == 0)
        def _():
            pltpu.sync_copy(acc.at[pl.ds(r0, ROWS_PER_TILE)], out_a.at[pl.ds(r0, ROWS_PER_TILE)])

        @pl.when(cid == 1)
        def _():
            pltpu.sync_copy(acc.at[pl.ds(r0, ROWS_PER_TILE)], out_b.at[pl.ds(r0, ROWS_PER_TILE)])

    return seg


_seg144 = _make_seg_sum(144)
_seg128 = _make_seg_sum(128)


def _ln(x, g, b, eps):
    m = x.mean(axis=-1, keepdims=True)
    v = ((x - m) ** 2).mean(axis=-1, keepdims=True)
    return g * (x - m) / jnp.sqrt(v + eps) + b


def _gnorm(x, w, b, a, eps=1e-5):
    mean = x.mean(axis=0, keepdims=True)
    out = x - a * mean
    var = (out * out).mean(axis=0, keepdims=True)
    return w * out / jnp.sqrt(var + eps) + b


def kernel(x_res, positions, AA, edge_index_bb, edge_index_knn, params):
    p = params
    ei_a, ei_b = edge_index_bb, edge_index_knn
    g = lambda t: jax.nn.gelu(t, approximate=False)
    x0 = _ln(x_res, p['ln_g'], p['ln_b'], 1e-6)
    h = jax.nn.relu(positions @ p['pW1'].T + p['pb1'])
    h = jax.nn.relu(h @ p['pW2'].T + p['pb2'])
    h = jax.nn.relu(h @ p['pW3'].T + p['pb3'])
    pos_enc = h @ p['pW4'].T + p['pb4']
    xin = jnp.concatenate([x0, pos_enc], axis=1)
    x = _ln(xin, p['ln2_g'], p['ln2_b'], 1e-6)
    x = g(x @ p['iW1'].T + p['ib1'])
    x = g(x @ p['iW2'].T + p['ib2'])

    ones16 = jnp.ones((N, 16), jnp.float32)
    zeros144 = jnp.zeros((N, 144), jnp.float32)
    zeros128 = jnp.zeros((N, 128), jnp.float32)

    y_a = jnp.concatenate([x @ p['Wl_a1'].T, ones16], axis=1)
    y_b = jnp.concatenate([x @ p['Wl_b1'].T, ones16], axis=1)
    agg_a, agg_b = _seg144(y_a, ei_a, y_b, ei_b, zeros144)
    cnt_a = jnp.clip(agg_a[:, 128:129], 1.0)
    cnt_b = jnp.clip(agg_b[:, 128:129], 1.0)
    xa = agg_a[:, :128] / cnt_a + p['bl_a1'] + x @ p['Wr_a1'].T
    xb = agg_b[:, :128] / cnt_b + p['bl_b1'] + x @ p['Wr_b1'].T
    x = g(0.5 * (xa + xb))
    x = _gnorm(x, p['g1_w'], p['g1_b'], p['g1_a'])
    x1 = x

    y_a2 = x @ p['Wl_a2'].T
    y_b2 = x @ p['Wl_b2'].T
    agg_a2, agg_b2 = _seg128(y_a2, ei_a, y_b2, ei_b, zeros128)
    xa = agg_a2 / cnt_a + p['bl_a2'] + x @ p['Wr_a2'].T
    xb = agg_b2 / cnt_b + p['bl_b2'] + x @ p['Wr_b2'].T
    x = g(0.5 * (xa + xb))
    x = _gnorm(x, p['g2_w'], p['g2_b'], p['g2_a'])
    x2 = x

    x = jnp.concatenate([x1, x2], axis=1)
    x = g(x @ p['hW1'].T + p['hb1'])
    x = g(x @ p['hW2'].T + p['hb2'])
    x = jnp.concatenate([x, AA], axis=1)
    x = g(x @ p['oW1'].T + p['ob1'])
    x = g(x @ p['oW2'].T + p['ob2'])
    x = jnp.tanh(x @ p['oW3'].T + p['ob3'])
    cb = p['codebook']
    d = jnp.sum(x * x, axis=1, keepdims=True) + jnp.sum(cb * cb, axis=1)[None, :] - 2.0 * (x @ cb.T)
    idx = jnp.argmin(d, axis=1)
    q = jnp.take(cb, idx, axis=0)
    e_loss = jnp.mean((jax.lax.stop_gradient(q) - x) ** 2)
    q_loss = jnp.mean((q - jax.lax.stop_gradient(x)) ** 2)
    vq_loss = q_loss + 0.25 * e_loss
    z_q = x + jax.lax.stop_gradient(q - x)
    return z_q, vq_loss
